# baseline (speedup 1.0000x reference)
import functools

import jax
import jax.numpy as jnp
from jax import lax
from jax.experimental import pallas as pl
from jax.experimental.pallas import tpu as pltpu
from jax.experimental.pallas import tpu_sc as plsc

TIME_LEN = 12
NC, NS = 2, 16
NW = NC * NS
BPW = 4
NBUK = NW * BPW
BS = 88
NPAD = NBUK * BS
G = 64
NOFF = 160


def _lane16():
    return lax.iota(jnp.int32, 16)


def _offs_scalar(chunks, idx):
    ci = idx // 16
    ln = idx % 16
    acc = jnp.zeros((16,), jnp.int32)
    for k, c in enumerate(chunks):
        acc = acc + jnp.where(ci == k, c, 0)
    sel = jnp.where(_lane16() == ln, acc, 0)
    return jnp.max(sel, axis=0)


@functools.lru_cache(maxsize=None)
def _make_propagate(D):
    mesh = plsc.VectorSubcoreMesh(core_axis_name="c", subcore_axis_name="s")

    @functools.partial(
        pl.kernel,
        out_type=jax.ShapeDtypeStruct((NPAD * D,), jnp.float32),
        mesh=mesh,
        compiler_params=pltpu.CompilerParams(needs_layout_passes=False),
        scratch_types=[
            pltpu.VMEM((BS * D,), jnp.float32),
            pltpu.VMEM((G, D), jnp.float32),
            pltpu.VMEM((G, D), jnp.float32),
            pltpu.VMEM((G,), jnp.int32),
            pltpu.VMEM((G,), jnp.int32),
            pltpu.VMEM((G,), jnp.int32),
            pltpu.VMEM((G,), jnp.int32),
            pltpu.VMEM((G,), jnp.float32),
            pltpu.VMEM((G,), jnp.float32),
            pltpu.VMEM((G,), jnp.float32),
            pltpu.VMEM((NPAD,), jnp.float32),
            pltpu.VMEM((NOFF,), jnp.int32),
            pltpu.SemaphoreType.DMA,
            pltpu.SemaphoreType.DMA,
        ],
    )
    def prop(y, yd, srcs, dsts, ws, dis, offs, out,
             acc, rows0, rows1, idx0, idx1, dst0, dst1, w0, w1,
             nrmv, disv, offv, sem0, sem1):
        rows = (rows0, rows1)
        idxb = (idx0, idx1)
        dstb = (dst0, dst1)
        wb = (w0, w1)
        sems = (sem0, sem1)
        wid = lax.axis_index("s") * NC + lax.axis_index("c")
        pltpu.sync_copy(offs, offv)
        pltpu.sync_copy(dis, disv)
        chunks = [offv[pl.ds(16 * k, 16)] for k in range(NOFF // 16)]

        def fetch_meta(bank, e0):
            pltpu.sync_copy(srcs.at[pl.ds(e0, G)], idxb[bank])
            pltpu.sync_copy(dsts.at[pl.ds(e0, G)], dstb[bank])
            pltpu.sync_copy(ws.at[pl.ds(e0, G)], wb[bank])

        for i in range(BPW):
            b = wid * BPW + i
            lo = b * BS
            estart = _offs_scalar(chunks, b)
            eend = _offs_scalar(chunks, b + 1)
            astart = estart - lax.rem(estart, 8)
            ntrip = (eend - astart + (G - 1)) // G
            pltpu.sync_copy(yd.at[pl.ds(lo * D, BS * D)], acc)
            lo_v = jnp.full((16,), lo, jnp.int32)

            @pl.when(ntrip > 0)
            def _():
                fetch_meta(0, pl.multiple_of(astart, 8))
                pltpu.async_copy(y.at[idxb[0]], rows[0], sems[0])

            def pair(p, carry):
                for cur in range(2):
                    g = p * 2 + cur
                    nxt = 1 - cur

                    @pl.when(g < ntrip)
                    def _():
                        @pl.when(g + 1 < ntrip)
                        def _():
                            e1 = pl.multiple_of(astart + (g + 1) * G, 8)
                            fetch_meta(nxt, e1)
                            pltpu.async_copy(y.at[idxb[nxt]], rows[nxt], sems[nxt])

                        pltpu.make_async_copy(y.at[pl.ds(0, G)], rows[cur], sems[cur]).wait()

                        for t in range(G // 16):
                            sl = pl.ds(16 * t, 16)
                            s16 = idxb[cur][sl]
                            d16 = jnp.minimum(dstb[cur][sl], NPAD - 1)
                            nrmv[sl] = (plsc.load_gather(disv, [s16]) * wb[cur][sl]
                                        * plsc.load_gather(disv, [d16]))

                        def edge2(j2, c2):
                            for u in range(2):
                                j = j2 * 2 + u
                                jv = jnp.full((16,), j, jnp.int32)
                                dv = plsc.load_gather(dstb[cur], [jv]) - lo_v
                                nv = plsc.load_gather(nrmv, [jv])
                                msk = (dv >= 0) & (dv < BS)
                                dvc = jnp.minimum(jnp.maximum(dv, 0), BS - 1)
                                bidx = dvc * D + _lane16()
                                for kk in range(D // 16):
                                    ch = rows[cur][j, pl.ds(kk * 16, 16)]
                                    plsc.addupdate_scatter(acc, [bidx + (kk * 16)], nv * ch, mask=msk)
                            return c2

                        lax.fori_loop(0, G // 2, edge2, 0)
                return carry

            lax.fori_loop(0, (ntrip + 1) // 2, pair, 0)
            pltpu.sync_copy(acc, out.at[pl.ds(lo * D, BS * D)])

    return prop


def _conv1d(x, w, b):
    out = jax.lax.conv_general_dilated(x, w, (1,), 'VALID', dimension_numbers=('NCH', 'OIH', 'NCH'))
    return out + b[None, :, None]


def _time_conv(x, w1, b1, w2, b2):
    return jnp.tanh(_conv1d(x, w1, b1)) * jax.nn.sigmoid(_conv1d(x, w2, b2))


def _st_block(xpad, routing, nd, P, pfx, n, k=3):
    src_sp, dst_sp, wp_sp, wn_sp, dis_p, dis_n, invdeg_p, invdeg_n, offs = routing
    out1 = _time_conv(xpad, P[pfx + 'tc1a_w'], P[pfx + 'tc1a_b'], P[pfx + 'tc1b_w'], P[pfx + 'tc1b_b'])
    d0, d1, d2 = out1.shape
    D = d1 * d2
    prop = _make_propagate(D)
    h = jax.nn.relu(nd @ P[pfx + 'psi1_W'].T + P[pfx + 'psi1_b'])
    psi = h @ P[pfx + 'psi2_W'].T + P[pfx + 'psi2_b']
    psi = jnp.pad(psi, ((0, NPAD - n), (0, 0)))

    def gcn(flat, W, bvec, w_sorted, dis, invdeg):
        y = flat @ W.T
        yd = (y * invdeg[:, None]).reshape(-1)
        agg = prop(y, yd, src_sp, dst_sp, w_sorted, dis, offs).reshape(NPAD, D)
        return agg + bvec

    cur = out1
    out_pos_psi = None
    for i in range(k):
        flat = cur.reshape(d0, -1)
        out_pos = jax.nn.relu(gcn(flat, P[pfx + 'gcnp_W'], P[pfx + 'gcnp_b'], wp_sp, dis_p, invdeg_p).reshape(d0, d1, d2))
        term = psi[:, i][:, None, None] * out_pos
        out_pos_psi = term if out_pos_psi is None else out_pos_psi + term
        cur = out_pos
    out_neg = jax.nn.relu(gcn(out1.reshape(d0, -1), P[pfx + 'gcnn_W'], P[pfx + 'gcnn_b'], wn_sp, dis_n, invdeg_n).reshape(d0, d1, d2))
    out2 = jnp.concatenate([out_pos_psi, out_neg], axis=1)
    out2 = jax.nn.relu(jnp.einsum('ncl,oc->nol', out2, P[pfx + 'gre_W']) + P[pfx + 'gre_b'][None, :, None])
    return _time_conv(out2, P[pfx + 'tc2a_w'], P[pfx + 'tc2a_b'], P[pfx + 'tc2b_w'], P[pfx + 'tc2b_b'])


def kernel(x, edge_index, edge_attr, params):
    P = params
    n = x.shape[0]
    e = edge_index.shape[1]
    epad = e + 128
    xt = x[:, :TIME_LEN]
    nd = x[:, TIME_LEN:]
    src = edge_index[0]
    dst = edge_index[1]

    wp = edge_attr[:, 0] + 1.0
    wn = edge_attr[:, 1] + 1.0
    dst_s, src_s, wp_s, wn_s = lax.sort((dst, src, wp, wn), num_keys=1)
    deg_p = jnp.zeros((n,), jnp.float32).at[dst].add(wp) + 1.0
    deg_n = jnp.zeros((n,), jnp.float32).at[dst].add(wn) + 1.0
    dis_p = jnp.pad(deg_p ** -0.5, (0, NPAD - n))
    dis_n = jnp.pad(deg_n ** -0.5, (0, NPAD - n))
    invdeg_p = jnp.pad(1.0 / deg_p, (0, NPAD - n))
    invdeg_n = jnp.pad(1.0 / deg_n, (0, NPAD - n))
    offs = jnp.searchsorted(dst_s, jnp.arange(NBUK + 1, dtype=jnp.int32) * BS).astype(jnp.int32)
    offs = jnp.pad(offs, (0, NOFF - (NBUK + 1)), constant_values=e)
    src_sp = jnp.pad(src_s, (0, epad - e))
    dst_sp = jnp.pad(dst_s, (0, epad - e), constant_values=NPAD)
    wp_sp = jnp.pad(wp_s, (0, epad - e))
    wn_sp = jnp.pad(wn_s, (0, epad - e))
    routing_p = (src_sp, dst_sp, wp_sp, wn_sp, dis_p, dis_n, invdeg_p, invdeg_n, offs)

    out = _conv1d(xt[:, None, :], P['fl_w'], P['fl_b'])
    out = jnp.pad(out, ((0, NPAD - n), (0, 0), (0, 0)))
    out = _st_block(out, routing_p, nd, P, 'b1_', n)
    out = _st_block(out, routing_p, nd, P, 'b2_', n)
    out = out[:n]
    out = _conv1d(out, P['out_conv_w'], P['out_conv_b'])[:, :, 0]
    out = jax.nn.relu(out)
    return out @ P['out_mlp_W'].T + P['out_mlp_b']


# speedup vs baseline: 1.2373x; 1.2373x over previous
import functools

import jax
import jax.numpy as jnp
from jax import lax
from jax.experimental import pallas as pl
from jax.experimental.pallas import tpu as pltpu
from jax.experimental.pallas import tpu_sc as plsc

TIME_LEN = 12
NC, NS = 2, 16
NW = NC * NS
BPW = 4
NBUK = NW * BPW
BS = 88
NPAD = NBUK * BS
G = 64
NOFF = 160


def _lane16():
    return lax.iota(jnp.int32, 16)


def _offs_scalar(chunks, idx):
    ci = idx // 16
    ln = idx % 16
    acc = jnp.zeros((16,), jnp.int32)
    for k, c in enumerate(chunks):
        acc = acc + jnp.where(ci == k, c, 0)
    sel = jnp.where(_lane16() == ln, acc, 0)
    return jnp.max(sel, axis=0)


@functools.lru_cache(maxsize=None)
def _make_propagate(D):
    mesh = plsc.VectorSubcoreMesh(core_axis_name="c", subcore_axis_name="s")

    @functools.partial(
        pl.kernel,
        out_type=jax.ShapeDtypeStruct((NPAD, D), jnp.float32),
        mesh=mesh,
        compiler_params=pltpu.CompilerParams(needs_layout_passes=False),
        scratch_types=[
            pltpu.VMEM((BS, D), jnp.float32),
            pltpu.VMEM((G, D), jnp.float32),
            pltpu.VMEM((G, D), jnp.float32),
            pltpu.VMEM((G,), jnp.int32),
            pltpu.VMEM((G,), jnp.int32),
            pltpu.VMEM((G,), jnp.int32),
            pltpu.VMEM((G,), jnp.int32),
            pltpu.VMEM((G,), jnp.float32),
            pltpu.VMEM((G,), jnp.float32),
            pltpu.VMEM((G,), jnp.float32),
            pltpu.VMEM((NPAD,), jnp.float32),
            pltpu.VMEM((NOFF,), jnp.int32),
            pltpu.SemaphoreType.DMA,
            pltpu.SemaphoreType.DMA,
        ],
    )
    def prop(y, yd, srcs, dsts, ws, dis, offs, out,
             acc, rows0, rows1, idx0, idx1, dst0, dst1, w0, w1,
             nrmv, disv, offv, sem0, sem1):
        rows = (rows0, rows1)
        idxb = (idx0, idx1)
        dstb = (dst0, dst1)
        wb = (w0, w1)
        sems = (sem0, sem1)
        wid = lax.axis_index("s") * NC + lax.axis_index("c")
        pltpu.sync_copy(offs, offv)
        pltpu.sync_copy(dis, disv)
        chunks = [offv[pl.ds(16 * k, 16)] for k in range(NOFF // 16)]

        def fetch_meta(bank, e0):
            pltpu.sync_copy(srcs.at[pl.ds(e0, G)], idxb[bank])
            pltpu.sync_copy(dsts.at[pl.ds(e0, G)], dstb[bank])
            pltpu.sync_copy(ws.at[pl.ds(e0, G)], wb[bank])

        for i in range(BPW):
            b = wid * BPW + i
            lo = b * BS
            estart = _offs_scalar(chunks, b)
            eend = _offs_scalar(chunks, b + 1)
            astart = estart - lax.rem(estart, 8)
            ntrip = (eend - astart + (G - 1)) // G
            pltpu.sync_copy(yd.at[pl.ds(lo, BS)], acc)
            lo_v = jnp.full((16,), lo, jnp.int32)

            @pl.when(ntrip > 0)
            def _():
                fetch_meta(0, pl.multiple_of(astart, 8))
                pltpu.async_copy(y.at[idxb[0]], rows[0], sems[0])

            def pair(p, carry):
                for cur in range(2):
                    g = p * 2 + cur
                    nxt = 1 - cur

                    @pl.when(g < ntrip)
                    def _():
                        @pl.when(g + 1 < ntrip)
                        def _():
                            e1 = pl.multiple_of(astart + (g + 1) * G, 8)
                            fetch_meta(nxt, e1)
                            pltpu.async_copy(y.at[idxb[nxt]], rows[nxt], sems[nxt])

                        pltpu.make_async_copy(y.at[pl.ds(0, G)], rows[cur], sems[cur]).wait()

                        for t in range(G // 16):
                            sl = pl.ds(16 * t, 16)
                            s16 = idxb[cur][sl]
                            d16 = jnp.minimum(dstb[cur][sl], NPAD - 1)
                            nrmv[sl] = (plsc.load_gather(disv, [s16]) * wb[cur][sl]
                                        * plsc.load_gather(disv, [d16]))

                        def edge(j2, c2):
                            j = (j2 % 8) * (G // 8) + j2 // 8
                            jv = jnp.full((16,), j, jnp.int32)
                            dv = plsc.load_gather(dstb[cur], [jv]) - lo_v
                            nv = plsc.load_gather(nrmv, [jv])
                            msk = (dv >= 0) & (dv < BS)
                            dvc = jnp.minimum(jnp.maximum(dv, 0), BS - 1)
                            for kk in range(D // 16):
                                colv = _lane16() + (kk * 16)
                                ch = rows[cur][j, pl.ds(kk * 16, 16)]
                                plsc.addupdate_scatter(acc, [dvc, colv], nv * ch, mask=msk)
                            return c2

                        lax.fori_loop(0, G, edge, 0)
                return carry

            lax.fori_loop(0, (ntrip + 1) // 2, pair, 0)
            pltpu.sync_copy(acc, out.at[pl.ds(lo, BS)])

    return prop


def _conv1d(x, w, b):
    out = jax.lax.conv_general_dilated(x, w, (1,), 'VALID', dimension_numbers=('NCH', 'OIH', 'NCH'))
    return out + b[None, :, None]


def _time_conv(x, w1, b1, w2, b2):
    return jnp.tanh(_conv1d(x, w1, b1)) * jax.nn.sigmoid(_conv1d(x, w2, b2))


def _st_block(xpad, routing, nd, P, pfx, n, k=3):
    src_sp, dst_sp, wp_sp, wn_sp, dis_p, dis_n, invdeg_p, invdeg_n, offs = routing
    out1 = _time_conv(xpad, P[pfx + 'tc1a_w'], P[pfx + 'tc1a_b'], P[pfx + 'tc1b_w'], P[pfx + 'tc1b_b'])
    d0, d1, d2 = out1.shape
    D = d1 * d2
    prop = _make_propagate(D)
    h = jax.nn.relu(nd @ P[pfx + 'psi1_W'].T + P[pfx + 'psi1_b'])
    psi = h @ P[pfx + 'psi2_W'].T + P[pfx + 'psi2_b']
    psi = jnp.pad(psi, ((0, NPAD - n), (0, 0)))

    def gcn(flat, W, bvec, w_sorted, dis, invdeg):
        y = flat @ W.T
        yd = y * invdeg[:, None]
        agg = prop(y, yd, src_sp, dst_sp, w_sorted, dis, offs)
        return agg + bvec

    cur = out1
    out_pos_psi = None
    for i in range(k):
        flat = cur.reshape(d0, -1)
        out_pos = jax.nn.relu(gcn(flat, P[pfx + 'gcnp_W'], P[pfx + 'gcnp_b'], wp_sp, dis_p, invdeg_p).reshape(d0, d1, d2))
        term = psi[:, i][:, None, None] * out_pos
        out_pos_psi = term if out_pos_psi is None else out_pos_psi + term
        cur = out_pos
    out_neg = jax.nn.relu(gcn(out1.reshape(d0, -1), P[pfx + 'gcnn_W'], P[pfx + 'gcnn_b'], wn_sp, dis_n, invdeg_n).reshape(d0, d1, d2))
    out2 = jnp.concatenate([out_pos_psi, out_neg], axis=1)
    out2 = jax.nn.relu(jnp.einsum('ncl,oc->nol', out2, P[pfx + 'gre_W']) + P[pfx + 'gre_b'][None, :, None])
    return _time_conv(out2, P[pfx + 'tc2a_w'], P[pfx + 'tc2a_b'], P[pfx + 'tc2b_w'], P[pfx + 'tc2b_b'])


def kernel(x, edge_index, edge_attr, params):
    P = params
    n = x.shape[0]
    e = edge_index.shape[1]
    epad = e + 128
    xt = x[:, :TIME_LEN]
    nd = x[:, TIME_LEN:]
    src = edge_index[0]
    dst = edge_index[1]

    wp = edge_attr[:, 0] + 1.0
    wn = edge_attr[:, 1] + 1.0
    dst_s, src_s, wp_s, wn_s = lax.sort((dst, src, wp, wn), num_keys=1)
    deg_p = jnp.zeros((n,), jnp.float32).at[dst].add(wp) + 1.0
    deg_n = jnp.zeros((n,), jnp.float32).at[dst].add(wn) + 1.0
    dis_p = jnp.pad(deg_p ** -0.5, (0, NPAD - n))
    dis_n = jnp.pad(deg_n ** -0.5, (0, NPAD - n))
    invdeg_p = jnp.pad(1.0 / deg_p, (0, NPAD - n))
    invdeg_n = jnp.pad(1.0 / deg_n, (0, NPAD - n))
    offs = jnp.searchsorted(dst_s, jnp.arange(NBUK + 1, dtype=jnp.int32) * BS).astype(jnp.int32)
    offs = jnp.pad(offs, (0, NOFF - (NBUK + 1)), constant_values=e)
    src_sp = jnp.pad(src_s, (0, epad - e))
    dst_sp = jnp.pad(dst_s, (0, epad - e), constant_values=NPAD)
    wp_sp = jnp.pad(wp_s, (0, epad - e))
    wn_sp = jnp.pad(wn_s, (0, epad - e))
    routing_p = (src_sp, dst_sp, wp_sp, wn_sp, dis_p, dis_n, invdeg_p, invdeg_n, offs)

    out = _conv1d(xt[:, None, :], P['fl_w'], P['fl_b'])
    out = jnp.pad(out, ((0, NPAD - n), (0, 0), (0, 0)))
    out = _st_block(out, routing_p, nd, P, 'b1_', n)
    out = _st_block(out, routing_p, nd, P, 'b2_', n)
    out = out[:n]
    out = _conv1d(out, P['out_conv_w'], P['out_conv_b'])[:, :, 0]
    out = jax.nn.relu(out)
    return out @ P['out_mlp_W'].T + P['out_mlp_b']


# speedup vs baseline: 1.2582x; 1.0170x over previous
import functools

import jax
import jax.numpy as jnp
from jax import lax
from jax.experimental import pallas as pl
from jax.experimental.pallas import tpu as pltpu
from jax.experimental.pallas import tpu_sc as plsc

TIME_LEN = 12
NC, NS = 2, 16
NW = NC * NS
BPW = 4
NBUK = NW * BPW
BS = 88
NPAD = NBUK * BS
NOFF = 160


def _lane16():
    return lax.iota(jnp.int32, 16)


def _offs_scalar(chunks, idx):
    ci = idx // 16
    ln = idx % 16
    acc = jnp.zeros((16,), jnp.int32)
    for k, c in enumerate(chunks):
        acc = acc + jnp.where(ci == k, c, 0)
    sel = jnp.where(_lane16() == ln, acc, 0)
    return jnp.max(sel, axis=0)


@functools.lru_cache(maxsize=None)
def _make_propagate(D):
    G = 64 if D > 256 else 128
    mesh = plsc.VectorSubcoreMesh(core_axis_name="c", subcore_axis_name="s")

    @functools.partial(
        pl.kernel,
        out_type=jax.ShapeDtypeStruct((NPAD, D), jnp.float32),
        mesh=mesh,
        compiler_params=pltpu.CompilerParams(needs_layout_passes=False),
        scratch_types=[
            pltpu.VMEM((BS, D), jnp.float32),
            pltpu.VMEM((G, D), jnp.float32),
            pltpu.VMEM((G, D), jnp.float32),
            pltpu.VMEM((G,), jnp.int32),
            pltpu.VMEM((G,), jnp.int32),
            pltpu.VMEM((G,), jnp.int32),
            pltpu.VMEM((G,), jnp.int32),
            pltpu.VMEM((G,), jnp.float32),
            pltpu.VMEM((G,), jnp.float32),
            pltpu.VMEM((G,), jnp.float32),
            pltpu.VMEM((NPAD,), jnp.float32),
            pltpu.VMEM((NOFF,), jnp.int32),
            pltpu.SemaphoreType.DMA,
            pltpu.SemaphoreType.DMA,
        ],
    )
    def prop(y, yd, srcs, dsts, ws, dis, offs, out,
             acc, rows0, rows1, idx0, idx1, dst0, dst1, w0, w1,
             nrmv, disv, offv, sem0, sem1):
        rows = (rows0, rows1)
        idxb = (idx0, idx1)
        dstb = (dst0, dst1)
        wb = (w0, w1)
        sems = (sem0, sem1)
        wid = lax.axis_index("s") * NC + lax.axis_index("c")
        pltpu.sync_copy(offs, offv)
        pltpu.sync_copy(dis, disv)
        chunks = [offv[pl.ds(16 * k, 16)] for k in range(NOFF // 16)]

        def fetch_meta(bank, e0):
            pltpu.sync_copy(srcs.at[pl.ds(e0, G)], idxb[bank])
            pltpu.sync_copy(dsts.at[pl.ds(e0, G)], dstb[bank])
            pltpu.sync_copy(ws.at[pl.ds(e0, G)], wb[bank])

        for i in range(BPW):
            b = wid * BPW + i
            lo = b * BS
            estart = _offs_scalar(chunks, b)
            eend = _offs_scalar(chunks, b + 1)
            astart = estart - lax.rem(estart, 8)
            ntrip = (eend - astart + (G - 1)) // G
            pltpu.sync_copy(yd.at[pl.ds(lo, BS)], acc)
            lo_v = jnp.full((16,), lo, jnp.int32)

            @pl.when(ntrip > 0)
            def _():
                fetch_meta(0, pl.multiple_of(astart, 8))
                pltpu.async_copy(y.at[idxb[0]], rows[0], sems[0])

            def pair(p, carry):
                for cur in range(2):
                    g = p * 2 + cur
                    nxt = 1 - cur

                    @pl.when(g < ntrip)
                    def _():
                        @pl.when(g + 1 < ntrip)
                        def _():
                            e1 = pl.multiple_of(astart + (g + 1) * G, 8)
                            fetch_meta(nxt, e1)
                            pltpu.async_copy(y.at[idxb[nxt]], rows[nxt], sems[nxt])

                        pltpu.make_async_copy(y.at[pl.ds(0, G)], rows[cur], sems[cur]).wait()

                        for t in range(G // 16):
                            sl = pl.ds(16 * t, 16)
                            s16 = idxb[cur][sl]
                            d16 = jnp.minimum(dstb[cur][sl], NPAD - 1)
                            nrmv[sl] = (plsc.load_gather(disv, [s16]) * wb[cur][sl]
                                        * plsc.load_gather(disv, [d16]))

                        def edge(j, c2):
                            jv = jnp.full((16,), j, jnp.int32)
                            dv = plsc.load_gather(dstb[cur], [jv]) - lo_v
                            nv = plsc.load_gather(nrmv, [jv])
                            msk = (dv >= 0) & (dv < BS)
                            dvc = jnp.minimum(jnp.maximum(dv, 0), BS - 1)
                            for kk in range(D // 16):
                                colv = _lane16() + (kk * 16)
                                ch = rows[cur][j, pl.ds(kk * 16, 16)]
                                plsc.addupdate_scatter(acc, [dvc, colv], nv * ch, mask=msk)
                            return c2

                        lax.fori_loop(0, G, edge, 0)
                return carry

            lax.fori_loop(0, (ntrip + 1) // 2, pair, 0)
            pltpu.sync_copy(acc, out.at[pl.ds(lo, BS)])

    return prop


def _conv1d(x, w, b):
    out = jax.lax.conv_general_dilated(x, w, (1,), 'VALID', dimension_numbers=('NCH', 'OIH', 'NCH'))
    return out + b[None, :, None]


def _time_conv(x, w1, b1, w2, b2):
    return jnp.tanh(_conv1d(x, w1, b1)) * jax.nn.sigmoid(_conv1d(x, w2, b2))


def _st_block(xpad, routing, nd, P, pfx, n, k=3):
    src_sp, dst_sp, wp_sp, wn_sp, dis_p, dis_n, invdeg_p, invdeg_n, offs = routing
    out1 = _time_conv(xpad, P[pfx + 'tc1a_w'], P[pfx + 'tc1a_b'], P[pfx + 'tc1b_w'], P[pfx + 'tc1b_b'])
    d0, d1, d2 = out1.shape
    D = d1 * d2
    prop = _make_propagate(D)
    h = jax.nn.relu(nd @ P[pfx + 'psi1_W'].T + P[pfx + 'psi1_b'])
    psi = h @ P[pfx + 'psi2_W'].T + P[pfx + 'psi2_b']
    psi = jnp.pad(psi, ((0, NPAD - n), (0, 0)))

    def gcn(flat, W, bvec, w_sorted, dis, invdeg):
        y = flat @ W.T
        yd = y * invdeg[:, None]
        agg = prop(y, yd, src_sp, dst_sp, w_sorted, dis, offs)
        return agg + bvec

    cur = out1
    out_pos_psi = None
    for i in range(k):
        flat = cur.reshape(d0, -1)
        out_pos = jax.nn.relu(gcn(flat, P[pfx + 'gcnp_W'], P[pfx + 'gcnp_b'], wp_sp, dis_p, invdeg_p).reshape(d0, d1, d2))
        term = psi[:, i][:, None, None] * out_pos
        out_pos_psi = term if out_pos_psi is None else out_pos_psi + term
        cur = out_pos
    out_neg = jax.nn.relu(gcn(out1.reshape(d0, -1), P[pfx + 'gcnn_W'], P[pfx + 'gcnn_b'], wn_sp, dis_n, invdeg_n).reshape(d0, d1, d2))
    out2 = jnp.concatenate([out_pos_psi, out_neg], axis=1)
    out2 = jax.nn.relu(jnp.einsum('ncl,oc->nol', out2, P[pfx + 'gre_W']) + P[pfx + 'gre_b'][None, :, None])
    return _time_conv(out2, P[pfx + 'tc2a_w'], P[pfx + 'tc2a_b'], P[pfx + 'tc2b_w'], P[pfx + 'tc2b_b'])


def kernel(x, edge_index, edge_attr, params):
    P = params
    n = x.shape[0]
    e = edge_index.shape[1]
    epad = e + 256
    xt = x[:, :TIME_LEN]
    nd = x[:, TIME_LEN:]
    src = edge_index[0]
    dst = edge_index[1]

    wp = edge_attr[:, 0] + 1.0
    wn = edge_attr[:, 1] + 1.0
    dst_s, src_s, wp_s, wn_s = lax.sort((dst, src, wp, wn), num_keys=1)
    deg_p = jnp.zeros((n,), jnp.float32).at[dst].add(wp) + 1.0
    deg_n = jnp.zeros((n,), jnp.float32).at[dst].add(wn) + 1.0
    dis_p = jnp.pad(deg_p ** -0.5, (0, NPAD - n))
    dis_n = jnp.pad(deg_n ** -0.5, (0, NPAD - n))
    invdeg_p = jnp.pad(1.0 / deg_p, (0, NPAD - n))
    invdeg_n = jnp.pad(1.0 / deg_n, (0, NPAD - n))
    offs = jnp.searchsorted(dst_s, jnp.arange(NBUK + 1, dtype=jnp.int32) * BS).astype(jnp.int32)
    offs = jnp.pad(offs, (0, NOFF - (NBUK + 1)), constant_values=e)
    src_sp = jnp.pad(src_s, (0, epad - e))
    dst_sp = jnp.pad(dst_s, (0, epad - e), constant_values=NPAD)
    wp_sp = jnp.pad(wp_s, (0, epad - e))
    wn_sp = jnp.pad(wn_s, (0, epad - e))
    routing_p = (src_sp, dst_sp, wp_sp, wn_sp, dis_p, dis_n, invdeg_p, invdeg_n, offs)

    out = _conv1d(xt[:, None, :], P['fl_w'], P['fl_b'])
    out = jnp.pad(out, ((0, NPAD - n), (0, 0), (0, 0)))
    out = _st_block(out, routing_p, nd, P, 'b1_', n)
    out = _st_block(out, routing_p, nd, P, 'b2_', n)
    out = out[:n]
    out = _conv1d(out, P['out_conv_w'], P['out_conv_b'])[:, :, 0]
    out = jax.nn.relu(out)
    return out @ P['out_mlp_W'].T + P['out_mlp_b']
